# baseline (speedup 1.0000x reference)
import jax
import jax.numpy as jnp
from jax import lax
from jax.experimental import pallas as pl
from jax.experimental.pallas import tpu as pltpu
from jax.experimental.pallas import tpu_sc as plsc

B = 16384
VOCAB = 100000
EMB_DIMS = [128, 64, 128, 128]
COL_OFF = [0, 128, 192, 320]
N_CONT = 16
CONT_OFF = 448
PAD_OFF = 464
N_PAD = 48
M_LENGTH = 512
N_EMB = sum(EMB_DIMS)
IN_DIM = N_EMB + N_CONT
K_PAD = 512

NC, NS = 2, 16
NW = NC * NS
ROWS_PER_W = B // NW
GCHUNK = 128
NCHUNK = ROWS_PER_W // GCHUNK
LANES = 16


def _sc_gather_body(xcat_t, xc, zpad, e0, e1, e2, e3, o,
                    idx_v, xc_v, zpad_v, buf_a, buf_b, gsems, wsems):
    wid = lax.axis_index("s") * NC + lax.axis_index("c")
    base = wid * ROWS_PER_W
    tables = (e0, e1, e2, e3)
    bufs = (buf_a, buf_b, buf_a, buf_a)

    def wb_chunk_copy(f, c):
        return pltpu.make_async_copy(
            bufs[f].at[pl.ds(c * GCHUNK, GCHUNK)],
            o.at[pl.ds(base + c * GCHUNK, GCHUNK),
                 pl.ds(COL_OFF[f], EMB_DIMS[f])],
            wsems.at[f],
        )

    pltpu.sync_copy(xc.at[pl.ds(base, ROWS_PER_W)], xc_v)
    pltpu.async_copy(
        xc_v, o.at[pl.ds(base, ROWS_PER_W), pl.ds(CONT_OFF, N_CONT)],
        wsems.at[4],
    )
    pltpu.sync_copy(zpad, zpad_v)
    for c in range(NCHUNK):
        pltpu.async_copy(
            zpad_v, o.at[pl.ds(base + c * GCHUNK, GCHUNK), pl.ds(PAD_OFF, N_PAD)],
            wsems.at[5],
        )

    for f in range(4):
        if f in (2, 3):
            for c in range(NCHUNK):
                wb_chunk_copy(f - 2, c).wait()
        pltpu.sync_copy(xcat_t.at[f, pl.ds(base, ROWS_PER_W)], idx_v)
        for c in range(NCHUNK):
            pltpu.async_copy(
                tables[f].at[idx_v.at[pl.ds(c * GCHUNK, GCHUNK)]],
                bufs[f].at[pl.ds(c * GCHUNK, GCHUNK)],
                gsems.at[c],
            )
        for c in range(NCHUNK):
            pltpu.make_async_copy(
                tables[f].at[idx_v.at[pl.ds(c * GCHUNK, GCHUNK)]],
                bufs[f].at[pl.ds(c * GCHUNK, GCHUNK)],
                gsems.at[c],
            ).wait()
            wb_chunk_copy(f, c).start()
    for f in (2, 3):
        for c in range(NCHUNK):
            wb_chunk_copy(f, c).wait()
    pltpu.make_async_copy(
        xc_v, o.at[pl.ds(base, ROWS_PER_W), pl.ds(CONT_OFF, N_CONT)],
        wsems.at[4],
    ).wait()
    for c in range(NCHUNK):
        pltpu.make_async_copy(
            zpad_v, o.at[pl.ds(base + c * GCHUNK, GCHUNK), pl.ds(PAD_OFF, N_PAD)],
            wsems.at[5],
        ).wait()


@jax.jit
def _sc_gather(xcat_t, xc, zpad, e0, e1, e2, e3):
    mesh = plsc.VectorSubcoreMesh(core_axis_name="c", subcore_axis_name="s")
    return pl.kernel(
        _sc_gather_body,
        out_type=jax.ShapeDtypeStruct((B, K_PAD), jnp.float32),
        mesh=mesh,
        scratch_types=[
            pltpu.VMEM((ROWS_PER_W,), jnp.int32),
            pltpu.VMEM((ROWS_PER_W, N_CONT), jnp.float32),
            pltpu.VMEM((GCHUNK, N_PAD), jnp.float32),
            pltpu.VMEM((ROWS_PER_W, 128), jnp.float32),
            pltpu.VMEM((ROWS_PER_W, 64), jnp.float32),
            pltpu.SemaphoreType.DMA((NCHUNK,)),
            pltpu.SemaphoreType.DMA((6,)),
        ],
        compiler_params=pltpu.CompilerParams(use_tc_tiling_on_sc=False),
        name="emb_gather_sc",
    )(xcat_t, xc, zpad, e0, e1, e2, e3)


def _tc_prep_body(xc, w, gamma, beta, w_eff, brow):
    xcf = xc[...]
    mean = jnp.mean(xcf, axis=0, keepdims=True)
    var = jnp.mean((xcf - mean) ** 2, axis=0, keepdims=True)
    scale = gamma[...] / jnp.sqrt(var + 1e-5)
    shift = beta[...] - mean * scale
    kpos = lax.broadcasted_iota(jnp.int32, (M_LENGTH, K_PAD), 1)
    wclean = jnp.where(kpos < IN_DIM, w[...], 0.0)
    brow[...] = jnp.ones((1, K_PAD), jnp.float32)
    brow[0:1, CONT_OFF:IN_DIM] = scale
    w_eff[...] = wclean * brow[...]
    brow[...] = jnp.zeros((1, K_PAD), jnp.float32)
    brow[0:1, CONT_OFF:IN_DIM] = shift
    brow[...] = lax.dot_general(brow[...], wclean, (((1,), (1,)), ((), ())),
                                preferred_element_type=jnp.float32,
                                precision=lax.Precision.HIGHEST)


@jax.jit
def _tc_prep(xc, w1, gamma, beta):
    whole = lambda s: pl.BlockSpec(s, lambda i: (0, 0))
    return pl.pallas_call(
        _tc_prep_body,
        grid=(1,),
        in_specs=[
            whole((B, N_CONT)),
            whole((M_LENGTH, K_PAD)),
            whole((1, N_CONT)),
            whole((1, N_CONT)),
        ],
        out_specs=[
            whole((M_LENGTH, K_PAD)),
            whole((1, K_PAD)),
        ],
        out_shape=[
            jax.ShapeDtypeStruct((M_LENGTH, K_PAD), jnp.float32),
            jax.ShapeDtypeStruct((1, K_PAD), jnp.float32),
        ],
        name="bn_fold_prep_tc",
    )(xc, w1, gamma, beta)


ROW_BLK = 2048
N_BLK = B // ROW_BLK


def _tc_body(x, w_eff, brow, b, mask, out):
    acc = lax.dot_general(x[...], w_eff[...], (((1,), (1,)), ((), ())),
                          preferred_element_type=jnp.float32,
                          precision=lax.Precision.HIGHEST)
    out[...] = (acc + brow[...]) * mask[...] + b[...]


@jax.jit
def _tc_project(x, w_eff, brow, b1r, mask):
    return pl.pallas_call(
        _tc_body,
        grid=(N_BLK,),
        in_specs=[
            pl.BlockSpec((ROW_BLK, K_PAD), lambda i: (i, 0)),
            pl.BlockSpec((M_LENGTH, K_PAD), lambda i: (0, 0)),
            pl.BlockSpec((1, K_PAD), lambda i: (0, 0)),
            pl.BlockSpec((1, M_LENGTH), lambda i: (0, 0)),
            pl.BlockSpec((ROW_BLK, 1), lambda i: (i, 0)),
        ],
        out_specs=pl.BlockSpec((ROW_BLK, M_LENGTH), lambda i: (i, 0)),
        out_shape=jax.ShapeDtypeStruct((B, M_LENGTH), jnp.float32),
        name="bn_mask_proj_tc",
    )(x, w_eff, brow, b1r, mask)


def kernel(x_categorical, x_continuous, emb0, emb1, emb2, emb3, W1, b1,
           bn_gamma, bn_beta):
    zpad = jnp.zeros((GCHUNK, N_PAD), jnp.float32)
    xcat_t = x_categorical.T.reshape(4, B)
    w_eff, brow = _tc_prep(
        x_continuous, W1,
        bn_gamma.reshape(1, N_CONT), bn_beta.reshape(1, N_CONT),
    )
    x = _sc_gather(xcat_t, x_continuous, zpad, emb0, emb1, emb2, emb3)
    vec = jax.random.uniform(jax.random.key(42), (B, 1))
    mask = (vec > 0.1).astype(jnp.float32)
    return _tc_project(x, w_eff, brow, b1.reshape(1, M_LENGTH), mask)


# speedup vs baseline: 1.1728x; 1.1728x over previous
import jax
import jax.numpy as jnp
from jax import lax
from jax.experimental import pallas as pl
from jax.experimental.pallas import tpu as pltpu
from jax.experimental.pallas import tpu_sc as plsc

B = 16384
VOCAB = 100000
EMB_DIMS = [128, 64, 128, 128]
COL_OFF = [0, 128, 192, 320]
N_CONT = 16
CONT_OFF = 448
PAD_OFF = 464
N_PAD = 48
M_LENGTH = 512
N_EMB = sum(EMB_DIMS)
IN_DIM = N_EMB + N_CONT
K_PAD = 512

NC, NS = 2, 16
NW = NC * NS
ROWS_PER_W = B // NW
GCHUNK = 128
NCHUNK = ROWS_PER_W // GCHUNK
LANES = 16


def _sc_gather_body(xcat_t, xc, zpad, e0, e1, e2, e3, o,
                    idx_v, xc_v, zpad_v, buf_a, buf_b, gsems, wsems):
    wid = lax.axis_index("s") * NC + lax.axis_index("c")
    base = wid * ROWS_PER_W
    tables = (e0, e1, e2, e3)
    bufs = (buf_a, buf_b, buf_a, buf_a)

    def wb_chunk_copy(f, c):
        return pltpu.make_async_copy(
            bufs[f].at[pl.ds(c * GCHUNK, GCHUNK)],
            o.at[pl.ds(base + c * GCHUNK, GCHUNK),
                 pl.ds(COL_OFF[f], EMB_DIMS[f])],
            wsems.at[f],
        )

    pltpu.sync_copy(xc.at[pl.ds(base, ROWS_PER_W)], xc_v)
    pltpu.async_copy(
        xc_v, o.at[pl.ds(base, ROWS_PER_W), pl.ds(CONT_OFF, N_CONT)],
        wsems.at[4],
    )
    pltpu.sync_copy(zpad, zpad_v)
    for c in range(NCHUNK):
        pltpu.async_copy(
            zpad_v, o.at[pl.ds(base + c * GCHUNK, GCHUNK), pl.ds(PAD_OFF, N_PAD)],
            wsems.at[5],
        )

    for f in range(4):
        if f in (2, 3):
            for c in range(NCHUNK):
                wb_chunk_copy(f - 2, c).wait()
        pltpu.sync_copy(xcat_t.at[f, pl.ds(base, ROWS_PER_W)], idx_v)
        for c in range(NCHUNK):
            pltpu.async_copy(
                tables[f].at[idx_v.at[pl.ds(c * GCHUNK, GCHUNK)]],
                bufs[f].at[pl.ds(c * GCHUNK, GCHUNK)],
                gsems.at[c],
            )
        for c in range(NCHUNK):
            pltpu.make_async_copy(
                tables[f].at[idx_v.at[pl.ds(c * GCHUNK, GCHUNK)]],
                bufs[f].at[pl.ds(c * GCHUNK, GCHUNK)],
                gsems.at[c],
            ).wait()
            wb_chunk_copy(f, c).start()
    for f in (2, 3):
        for c in range(NCHUNK):
            wb_chunk_copy(f, c).wait()
    pltpu.make_async_copy(
        xc_v, o.at[pl.ds(base, ROWS_PER_W), pl.ds(CONT_OFF, N_CONT)],
        wsems.at[4],
    ).wait()
    for c in range(NCHUNK):
        pltpu.make_async_copy(
            zpad_v, o.at[pl.ds(base + c * GCHUNK, GCHUNK), pl.ds(PAD_OFF, N_PAD)],
            wsems.at[5],
        ).wait()


@jax.jit
def _sc_gather(xcat_t, xc, zpad, e0, e1, e2, e3):
    mesh = plsc.VectorSubcoreMesh(core_axis_name="c", subcore_axis_name="s")
    return pl.kernel(
        _sc_gather_body,
        out_type=jax.ShapeDtypeStruct((B, K_PAD), jnp.float32),
        mesh=mesh,
        scratch_types=[
            pltpu.VMEM((ROWS_PER_W,), jnp.int32),
            pltpu.VMEM((ROWS_PER_W, N_CONT), jnp.float32),
            pltpu.VMEM((GCHUNK, N_PAD), jnp.float32),
            pltpu.VMEM((ROWS_PER_W, 128), jnp.float32),
            pltpu.VMEM((ROWS_PER_W, 64), jnp.float32),
            pltpu.SemaphoreType.DMA((NCHUNK,)),
            pltpu.SemaphoreType.DMA((6,)),
        ],
        compiler_params=pltpu.CompilerParams(use_tc_tiling_on_sc=False),
        name="emb_gather_sc",
    )(xcat_t, xc, zpad, e0, e1, e2, e3)


def _tc_prep_body(xc, w, gamma, beta, w_eff, brow):
    xcf = xc[...]
    mean = jnp.mean(xcf, axis=0, keepdims=True)
    var = jnp.mean((xcf - mean) ** 2, axis=0, keepdims=True)
    scale = gamma[...] / jnp.sqrt(var + 1e-5)
    shift = beta[...] - mean * scale
    kpos = lax.broadcasted_iota(jnp.int32, (M_LENGTH, K_PAD), 1)
    wclean = jnp.where(kpos < IN_DIM, w[...], 0.0)
    brow[...] = jnp.ones((1, K_PAD), jnp.float32)
    brow[0:1, CONT_OFF:IN_DIM] = scale
    w_eff[...] = wclean * brow[...]
    brow[...] = jnp.zeros((1, K_PAD), jnp.float32)
    brow[0:1, CONT_OFF:IN_DIM] = shift
    brow[...] = lax.dot_general(brow[...], wclean, (((1,), (1,)), ((), ())),
                                preferred_element_type=jnp.float32,
                                precision=lax.Precision.HIGHEST)


@jax.jit
def _tc_prep(xc, w1, gamma, beta):
    whole = lambda s: pl.BlockSpec(s, lambda i: (0, 0))
    return pl.pallas_call(
        _tc_prep_body,
        grid=(1,),
        in_specs=[
            whole((B, N_CONT)),
            whole((M_LENGTH, K_PAD)),
            whole((1, N_CONT)),
            whole((1, N_CONT)),
        ],
        out_specs=[
            whole((M_LENGTH, K_PAD)),
            whole((1, K_PAD)),
        ],
        out_shape=[
            jax.ShapeDtypeStruct((M_LENGTH, K_PAD), jnp.float32),
            jax.ShapeDtypeStruct((1, K_PAD), jnp.float32),
        ],
        name="bn_fold_prep_tc",
    )(xc, w1, gamma, beta)


ROW_BLK = 2048
N_BLK = B // ROW_BLK


def _tc_body(x, w_eff, brow, b, mask, out):
    acc = lax.dot_general(x[...], w_eff[...], (((1,), (1,)), ((), ())),
                          preferred_element_type=jnp.float32,
                          precision=lax.Precision.DEFAULT)
    out[...] = (acc + brow[...]) * mask[...] + b[...]


@jax.jit
def _tc_project(x, w_eff, brow, b1r, mask):
    return pl.pallas_call(
        _tc_body,
        grid=(N_BLK,),
        in_specs=[
            pl.BlockSpec((ROW_BLK, K_PAD), lambda i: (i, 0)),
            pl.BlockSpec((M_LENGTH, K_PAD), lambda i: (0, 0)),
            pl.BlockSpec((1, K_PAD), lambda i: (0, 0)),
            pl.BlockSpec((1, M_LENGTH), lambda i: (0, 0)),
            pl.BlockSpec((ROW_BLK, 1), lambda i: (i, 0)),
        ],
        out_specs=pl.BlockSpec((ROW_BLK, M_LENGTH), lambda i: (i, 0)),
        out_shape=jax.ShapeDtypeStruct((B, M_LENGTH), jnp.float32),
        name="bn_mask_proj_tc",
    )(x, w_eff, brow, b1r, mask)


def kernel(x_categorical, x_continuous, emb0, emb1, emb2, emb3, W1, b1,
           bn_gamma, bn_beta):
    zpad = jnp.zeros((GCHUNK, N_PAD), jnp.float32)
    xcat_t = x_categorical.T.reshape(4, B)
    w_eff, brow = _tc_prep(
        x_continuous, W1,
        bn_gamma.reshape(1, N_CONT), bn_beta.reshape(1, N_CONT),
    )
    x = _sc_gather(xcat_t, x_continuous, zpad, emb0, emb1, emb2, emb3)
    vec = jax.random.uniform(jax.random.key(42), (B, 1))
    mask = (vec > 0.1).astype(jnp.float32)
    return _tc_project(x, w_eff, brow, b1.reshape(1, M_LENGTH), mask)


# speedup vs baseline: 1.2914x; 1.1012x over previous
import jax
import jax.numpy as jnp
import numpy as np
from jax import lax
from jax.experimental import pallas as pl
from jax.experimental.pallas import tpu as pltpu
from jax.experimental.pallas import tpu_sc as plsc

B = 16384
VOCAB = 100000
EMB_DIMS = [128, 64, 128, 128]
COL_OFF = [0, 128, 192, 320]
N_CONT = 16
CONT_OFF = 448
N_PAD = 64
M_LENGTH = 512
N_EMB = sum(EMB_DIMS)
IN_DIM = N_EMB + N_CONT
K_PAD = 512

NC, NS = 2, 16
NW = NC * NS
ROWS_PER_W = B // NW
GCHUNK = 128
NCHUNK = ROWS_PER_W // GCHUNK

_MASK_NP = np.asarray(
    jax.random.uniform(jax.random.key(42), (B, 1)) > 0.1, dtype=np.float32)


def _sc_gather_body(xcat_t, zpad, e0, e1, e2, e3, o,
                    idx_v, zpad_v, buf_a, buf_b, gsems, wsems):
    wid = lax.axis_index("s") * NC + lax.axis_index("c")
    base = wid * ROWS_PER_W
    tables = (e0, e1, e2, e3)
    bufs = (buf_a, buf_b, buf_a, buf_a)

    def wb_chunk_copy(f, c):
        return pltpu.make_async_copy(
            bufs[f].at[pl.ds(c * GCHUNK, GCHUNK)],
            o.at[pl.ds(base + c * GCHUNK, GCHUNK),
                 pl.ds(COL_OFF[f], EMB_DIMS[f])],
            wsems.at[f],
        )

    pltpu.sync_copy(zpad, zpad_v)
    for c in range(NCHUNK):
        pltpu.async_copy(
            zpad_v, o.at[pl.ds(base + c * GCHUNK, GCHUNK),
                         pl.ds(CONT_OFF, N_PAD)],
            wsems.at[4],
        )

    for f in range(4):
        if f in (2, 3):
            for c in range(NCHUNK):
                wb_chunk_copy(f - 2, c).wait()
        pltpu.sync_copy(xcat_t.at[f, pl.ds(base, ROWS_PER_W)], idx_v)
        for c in range(NCHUNK):
            pltpu.async_copy(
                tables[f].at[idx_v.at[pl.ds(c * GCHUNK, GCHUNK)]],
                bufs[f].at[pl.ds(c * GCHUNK, GCHUNK)],
                gsems.at[c],
            )
        for c in range(NCHUNK):
            pltpu.make_async_copy(
                tables[f].at[idx_v.at[pl.ds(c * GCHUNK, GCHUNK)]],
                bufs[f].at[pl.ds(c * GCHUNK, GCHUNK)],
                gsems.at[c],
            ).wait()
            wb_chunk_copy(f, c).start()
    for f in (2, 3):
        for c in range(NCHUNK):
            wb_chunk_copy(f, c).wait()
    for c in range(NCHUNK):
        pltpu.make_async_copy(
            zpad_v, o.at[pl.ds(base + c * GCHUNK, GCHUNK),
                         pl.ds(CONT_OFF, N_PAD)],
            wsems.at[4],
        ).wait()


@jax.jit
def _sc_gather(xcat_t, zpad, e0, e1, e2, e3):
    mesh = plsc.VectorSubcoreMesh(core_axis_name="c", subcore_axis_name="s")
    return pl.kernel(
        _sc_gather_body,
        out_type=jax.ShapeDtypeStruct((B, K_PAD), jnp.float32),
        mesh=mesh,
        scratch_types=[
            pltpu.VMEM((ROWS_PER_W,), jnp.int32),
            pltpu.VMEM((GCHUNK, N_PAD), jnp.float32),
            pltpu.VMEM((ROWS_PER_W, 128), jnp.float32),
            pltpu.VMEM((ROWS_PER_W, 64), jnp.float32),
            pltpu.SemaphoreType.DMA((NCHUNK,)),
            pltpu.SemaphoreType.DMA((5,)),
        ],
        compiler_params=pltpu.CompilerParams(use_tc_tiling_on_sc=False),
        name="emb_gather_sc",
    )(xcat_t, zpad, e0, e1, e2, e3)


def _tc_stats_body(xc, gamma, beta, scale_o, shift_o):
    xcf = xc[...]
    mean = jnp.mean(xcf, axis=0, keepdims=True)
    var = jnp.mean((xcf - mean) ** 2, axis=0, keepdims=True)
    scale = gamma[...] / jnp.sqrt(var + 1e-5)
    scale_o[...] = scale
    shift_o[...] = beta[...] - mean * scale


@jax.jit
def _tc_stats(xc, gamma, beta):
    whole = lambda s: pl.BlockSpec(s, lambda i: (0, 0))
    return pl.pallas_call(
        _tc_stats_body,
        grid=(1,),
        in_specs=[
            whole((B, N_CONT)),
            whole((1, N_CONT)),
            whole((1, N_CONT)),
        ],
        out_specs=[whole((1, N_CONT)), whole((1, N_CONT))],
        out_shape=[
            jax.ShapeDtypeStruct((1, N_CONT), jnp.float32),
            jax.ShapeDtypeStruct((1, N_CONT), jnp.float32),
        ],
        name="bn_stats_tc",
    )(xc, gamma, beta)


ROW_BLK = 2048
N_BLK = B // ROW_BLK


def _tc_body(x, xc, w, scale, shift, b, mask, out):
    i = pl.program_id(0)
    xcn = xc[pl.ds(i * ROW_BLK, ROW_BLK), :] * scale[...] + shift[...]
    kpos = lax.broadcasted_iota(jnp.int32, (M_LENGTH, K_PAD), 1)
    wfull = w[...]
    wmain = jnp.where(kpos < CONT_OFF, wfull, 0.0)
    wc = wfull[:, CONT_OFF:CONT_OFF + N_CONT]
    acc = lax.dot_general(x[...], wmain, (((1,), (1,)), ((), ())),
                          preferred_element_type=jnp.float32,
                          precision=lax.Precision.DEFAULT)
    acc += lax.dot_general(xcn, wc, (((1,), (1,)), ((), ())),
                           preferred_element_type=jnp.float32,
                           precision=lax.Precision.DEFAULT)
    out[...] = acc * mask[...] + b[...]


@jax.jit
def _tc_project(x, xc, w1, scale, shift, b1r, mask):
    return pl.pallas_call(
        _tc_body,
        grid=(N_BLK,),
        in_specs=[
            pl.BlockSpec((ROW_BLK, K_PAD), lambda i: (i, 0)),
            pl.BlockSpec((B, N_CONT), lambda i: (0, 0)),
            pl.BlockSpec((M_LENGTH, K_PAD), lambda i: (0, 0)),
            pl.BlockSpec((1, N_CONT), lambda i: (0, 0)),
            pl.BlockSpec((1, N_CONT), lambda i: (0, 0)),
            pl.BlockSpec((1, M_LENGTH), lambda i: (0, 0)),
            pl.BlockSpec((ROW_BLK, 1), lambda i: (i, 0)),
        ],
        out_specs=pl.BlockSpec((ROW_BLK, M_LENGTH), lambda i: (i, 0)),
        out_shape=jax.ShapeDtypeStruct((B, M_LENGTH), jnp.float32),
        name="bn_mask_proj_tc",
    )(x, xc, w1, scale, shift, b1r, mask)


def kernel(x_categorical, x_continuous, emb0, emb1, emb2, emb3, W1, b1,
           bn_gamma, bn_beta):
    xcat_t = x_categorical.T.reshape(4, B)
    zpad = jnp.zeros((GCHUNK, N_PAD), jnp.float32)
    x = _sc_gather(xcat_t, zpad, emb0, emb1, emb2, emb3)
    scale, shift = _tc_stats(
        x_continuous,
        bn_gamma.reshape(1, N_CONT), bn_beta.reshape(1, N_CONT),
    )
    mask = jnp.asarray(_MASK_NP)
    return _tc_project(x, x_continuous, W1, scale, shift,
                       b1.reshape(1, M_LENGTH), mask)
